# baseline (speedup 1.0000x reference)
import jax
import jax.numpy as jnp
from jax import lax
from jax.experimental import pallas as pl
from jax.experimental.pallas import tpu as pltpu
from jax.experimental.pallas import tpu_sc as plsc

GAMMA = 0.1
EPS = 0.1

N = 10000
E = 320000
D = 128

NC = 2
NS = 16
L = 16

NP = 10240
EP = 327680
CHUNK = 128
EROWS = EP // CHUNK
EPW = EP // (NC * NS)
NCHUNK = EPW // CHUNK
HROWS = EROWS // NS
IBUF = 40
SELF0 = EP - E
TROWS = 32

RPS = NP // NS
RCHUNK = 128
NRC = RPS // RCHUNK
DSLC = NP // NS


def _mm_body(x_ref, w_ref, o_ref):
    o_ref[...] = jnp.dot(x_ref[...], w_ref[...],
                         preferred_element_type=jnp.float32)


def _matmul(x, w):
    bm = 1024
    return pl.pallas_call(
        _mm_body,
        grid=(NP // bm,),
        in_specs=[pl.BlockSpec((bm, D), lambda i: (i, 0)),
                  pl.BlockSpec((D, D), lambda i: (0, 0))],
        out_specs=pl.BlockSpec((bm, D), lambda i: (i, 0)),
        out_shape=jax.ShapeDtypeStruct((NP, D), jnp.float32),
    )(x, w)


def _fin_body(x_ref, w_ref, b_ref, p_ref, o_ref):
    w = w_ref[...]
    ii = lax.broadcasted_iota(jnp.int32, (D, D), 0)
    jj = lax.broadcasted_iota(jnp.int32, (D, D), 1)
    gi = jnp.where(ii == jj, jnp.float32(GAMMA), jnp.float32(0.0))
    a = w.T - w - gi
    xa = jnp.dot(x_ref[...], a, preferred_element_type=jnp.float32)
    h = xa + p_ref[0] + p_ref[1] + b_ref[...]
    o_ref[...] = x_ref[...] + EPS * jnp.tanh(h)


def _finish(x, w, bias2d, partials):
    bm = 1024
    return pl.pallas_call(
        _fin_body,
        grid=(NP // bm,),
        in_specs=[pl.BlockSpec((bm, D), lambda i: (i, 0)),
                  pl.BlockSpec((D, D), lambda i: (0, 0)),
                  pl.BlockSpec((1, D), lambda i: (0, 0)),
                  pl.BlockSpec((NC, bm, D), lambda i: (0, i, 0))],
        out_specs=pl.BlockSpec((bm, D), lambda i: (i, 0)),
        out_shape=jax.ShapeDtypeStruct((N, D), jnp.float32),
    )(x, w, bias2d, partials)


def _rsqrt16(v):
    s = 0.5 * (v + 1.0)
    for _ in range(9):
        s = 0.5 * (s + v / s)
    return 1.0 / s


def _graph_body(xw_hbm, src_hbm, dst_hbm, tidx_hbm, out_hbm,
                y_hbm, deg_sh, acc_sh,
                sidx_v, didx_v, rbuf_v, gbuf_v,
                degv, dinvv, ones_v, sem, sem2, sem3, sem4):
    c = lax.axis_index("c")
    s = lax.axis_index("s")
    zero16 = jnp.zeros((L,), jnp.float32)

    with jax.named_scope("ph_a_zero"):
        for i in range(DSLC // L):
            degv[pl.ds(i * L, L)] = zero16

        def _zrow(r, _):
            for k in range(D // L):
                rbuf_v[r, pl.ds(k * L, L)] = zero16
            return 0

        lax.fori_loop(0, RCHUNK, _zrow, 0)
        pltpu.async_copy(degv.at[pl.ds(0, DSLC)], deg_sh.at[pl.ds(s * DSLC, DSLC)], sem)
        for t in range(NRC):
            pltpu.async_copy(rbuf_v,
                             acc_sh.at[pl.ds(s * RPS + t * RCHUNK, RCHUNK)],
                             sem)
        for i in range(CHUNK // L):
            ones_v[pl.ds(i * L, L)] = jnp.ones((L,), jnp.float32)
        pltpu.make_async_copy(degv.at[pl.ds(0, DSLC)],
                              deg_sh.at[pl.ds(s * DSLC, DSLC)], sem).wait()
        for t in range(NRC):
            pltpu.make_async_copy(
                rbuf_v, acc_sh.at[pl.ds(s * RPS + t * RCHUNK, RCHUNK)],
                sem).wait()
        plsc.subcore_barrier()

    with jax.named_scope("ph_b_hist"):
        def _hb_start(j):
            pltpu.async_copy(ones_v, deg_sh.at[sidx_v.at[j]], sem, add=True)

        def _hb_wait(j):
            pltpu.make_async_copy(ones_v, deg_sh.at[sidx_v.at[j]], sem).wait()

        def _hist(j, _):
            _hb_start(j)

            @pl.when(j >= 8)
            def _():
                _hb_wait(j - 8)

            return 0

        for h in range(HROWS // IBUF):
            pltpu.sync_copy(dst_hbm.at[pl.ds(s * HROWS + h * IBUF, IBUF)],
                            sidx_v)
            lax.fori_loop(0, IBUF, _hist, 0)
            for k in range(8):
                _hb_wait(k)
        t0 = (s // 4) * 8
        pltpu.sync_copy(tidx_hbm.at[pl.ds(t0, 8)], sidx_v.at[pl.ds(0, 8)])
        r0 = 2 * s - t0
        for q in range(2):
            pltpu.async_copy(ones_v, deg_sh.at[sidx_v.at[r0 + q]], sem,
                             add=True)
        for q in range(2):
            pltpu.make_async_copy(ones_v, deg_sh.at[sidx_v.at[r0 + q]],
                                  sem).wait()
        plsc.subcore_barrier()

    with jax.named_scope("ph_c_scale"):
        pltpu.sync_copy(deg_sh.at[pl.ds(s * DSLC, DSLC)], degv.at[pl.ds(0, DSLC)])
        for i in range(DSLC // L):
            dinvv[pl.ds(i * L, L)] = _rsqrt16(degv[pl.ds(i * L, L)])
        bufs = (rbuf_v, gbuf_v)
        sems = (sem, sem2)

        def _xw_start(t, b):
            pltpu.async_copy(xw_hbm.at[pl.ds(s * RPS + t * RCHUNK, RCHUNK)],
                             bufs[b], sems[b])

        def _xw_wait(t, b):
            pltpu.make_async_copy(
                xw_hbm.at[pl.ds(s * RPS + t * RCHUNK, RCHUNK)],
                bufs[b], sems[b]).wait()

        _xw_start(0, 0)
        _xw_start(1, 1)
        for t in range(NRC):
            b = t % 2
            buf = bufs[b]
            base = s * RPS + t * RCHUNK
            _xw_wait(t, b)

            def _scale(r, _, buf=buf, t=t):
                d = dinvv[pl.ds(t * RCHUNK + r, L)][0]
                for k in range(D // L):
                    sl = pl.ds(k * L, L)
                    buf[r, sl] = buf[r, sl] * d
                return 0

            lax.fori_loop(0, RCHUNK - 1, _scale, 0)
            dlast = dinvv[pl.ds(t * RCHUNK + RCHUNK - L, L)][L - 1]
            for k in range(D // L):
                sl = pl.ds(k * L, L)
                buf[RCHUNK - 1, sl] = buf[RCHUNK - 1, sl] * dlast
            pltpu.sync_copy(buf, y_hbm.at[c].at[pl.ds(base, RCHUNK)])
            if t + 2 < NRC:
                _xw_start(t + 2, b)
        plsc.subcore_barrier()

    with jax.named_scope("ph_d_edge"):
        row0 = (c * NS + s) * NCHUNK

        def _gstart(j, buf, gsem):
            pltpu.async_copy(y_hbm.at[c].at[sidx_v.at[j]], buf, gsem)

        def _gwait(j, buf, gsem):
            pltpu.make_async_copy(y_hbm.at[c].at[sidx_v.at[j]], buf,
                                  gsem).wait()

        def _sstart(j, buf, ssem):
            pltpu.async_copy(buf, acc_sh.at[didx_v.at[j]], ssem, add=True)

        def _swait(j, buf, ssem):
            pltpu.make_async_copy(buf, acc_sh.at[didx_v.at[j]], ssem).wait()

        for h in range(NCHUNK // IBUF):
            pltpu.sync_copy(src_hbm.at[pl.ds(row0 + h * IBUF, IBUF)], sidx_v)
            pltpu.sync_copy(dst_hbm.at[pl.ds(row0 + h * IBUF, IBUF)], didx_v)
            _gstart(0, gbuf_v, sem)
            _gstart(1, rbuf_v, sem2)

            def _edge2(jj, _):
                j0 = 2 * jj
                j1 = j0 + 1
                _gwait(j0, gbuf_v, sem)
                _sstart(j0, gbuf_v, sem3)
                _gwait(j1, rbuf_v, sem2)
                _sstart(j1, rbuf_v, sem4)
                _swait(j0, gbuf_v, sem3)
                _gstart(jnp.minimum(j0 + 2, IBUF - 1), gbuf_v, sem)
                _swait(j1, rbuf_v, sem4)
                _gstart(jnp.minimum(j1 + 2, IBUF - 1), rbuf_v, sem2)
                return 0

            lax.fori_loop(0, IBUF // 2, _edge2, 0)
            _gwait(IBUF - 1, gbuf_v, sem)
            _gwait(IBUF - 1, rbuf_v, sem2)
        w = c * NS + s
        tw = (w // 8) * 8
        pltpu.sync_copy(tidx_hbm.at[pl.ds(tw, 8)], sidx_v.at[pl.ds(0, 8)])
        rw = w - tw
        pltpu.async_copy(y_hbm.at[c].at[sidx_v.at[rw]], gbuf_v, sem)
        pltpu.make_async_copy(y_hbm.at[c].at[sidx_v.at[rw]], gbuf_v,
                              sem).wait()
        pltpu.sync_copy(gbuf_v, acc_sh.at[sidx_v.at[rw]], add=True)
        plsc.subcore_barrier()

    with jax.named_scope("ph_e_stage"):
        pltpu.async_copy(acc_sh.at[pl.ds(s * RPS, RCHUNK)], rbuf_v, sem)
        for t in range(NRC):
            base = s * RPS + t * RCHUNK
            pltpu.make_async_copy(acc_sh.at[pl.ds(base, RCHUNK)], rbuf_v,
                                  sem).wait()

            def _fin(r, _):
                d = dinvv[pl.ds(t * RCHUNK + r, L)][0]
                for k in range(D // L):
                    sl = pl.ds(k * L, L)
                    gbuf_v[r, sl] = d * rbuf_v[r, sl]
                return 0

            lax.fori_loop(0, RCHUNK - 1, _fin, 0)
            dlast = dinvv[pl.ds(t * RCHUNK + RCHUNK - L, L)][L - 1]
            for k in range(D // L):
                sl = pl.ds(k * L, L)
                gbuf_v[RCHUNK - 1, sl] = dlast * rbuf_v[RCHUNK - 1, sl]
            if t + 1 < NRC:
                pltpu.async_copy(acc_sh.at[pl.ds(base + RCHUNK, RCHUNK)],
                                 rbuf_v, sem)
            pltpu.sync_copy(gbuf_v, out_hbm.at[c].at[pl.ds(base, RCHUNK)])


def _graph_sc(xw, src2d, dst2d, tail2d):
    mesh = plsc.VectorSubcoreMesh(core_axis_name="c", subcore_axis_name="s")
    kfn = pl.kernel(
        _graph_body,
        out_type=jax.ShapeDtypeStruct((NC, NP, D), jnp.float32),
        mesh=mesh,
        scratch_types=[
            pltpu.HBM((NC, NP, D), jnp.float32),
            pltpu.VMEM_SHARED((NP,), jnp.float32),
            pltpu.VMEM_SHARED((NP, D), jnp.float32),
            pltpu.VMEM((IBUF, CHUNK), jnp.int32),
            pltpu.VMEM((IBUF, CHUNK), jnp.int32),
            pltpu.VMEM((RCHUNK, D), jnp.float32),
            pltpu.VMEM((CHUNK, D), jnp.float32),
            pltpu.VMEM((DSLC + L,), jnp.float32),
            pltpu.VMEM((DSLC + L,), jnp.float32),
            pltpu.VMEM((CHUNK,), jnp.float32),
            pltpu.SemaphoreType.DMA,
            pltpu.SemaphoreType.DMA,
            pltpu.SemaphoreType.DMA,
            pltpu.SemaphoreType.DMA,
        ],
    )
    return kfn(xw, src2d, dst2d, tail2d)


@jax.jit
def kernel(x, edge_index, W, bias, W_phi):
    self_idx = jnp.arange(SELF0, dtype=jnp.int32)
    src2d = jnp.concatenate([edge_index[0], self_idx]).reshape(EROWS, CHUNK)
    dst2d = jnp.concatenate([edge_index[1], self_idx]).reshape(EROWS, CHUNK)
    tail_self = SELF0 + jnp.arange(N - SELF0, dtype=jnp.int32)
    tail_pad = N + jnp.arange(TROWS * CHUNK - (N - SELF0),
                              dtype=jnp.int32) % (NP - N)
    tail2d = jnp.concatenate([tail_self, tail_pad]).reshape(TROWS, CHUNK)
    x_pad = jnp.pad(x, ((0, NP - N), (0, 0)))
    xw = _matmul(x_pad, W_phi)
    partials = _graph_sc(xw, src2d, dst2d, tail2d)
    return _finish(x_pad, W, bias.reshape(1, D), partials)


# speedup vs baseline: 1.2118x; 1.2118x over previous
import jax
import jax.numpy as jnp
from jax import lax
from jax.experimental import pallas as pl
from jax.experimental.pallas import tpu as pltpu
from jax.experimental.pallas import tpu_sc as plsc

GAMMA = 0.1
EPS = 0.1

N = 10000
E = 320000
D = 128

NC = 2
NS = 16
L = 16

NP = 10240
EP = 327680
CHUNK = 128
EROWS = EP // CHUNK
EPW = EP // (NC * NS)
NCHUNK = EPW // CHUNK
HROWS = EROWS // NS
IBUF = 40
SELF0 = EP - E
TROWS = 32

RPS = NP // NS
RCHUNK = 128
NRC = RPS // RCHUNK
DSLC = NP // NS


def _mm_body(x_ref, w_ref, o_ref):
    o_ref[...] = jnp.dot(x_ref[...], w_ref[...],
                         preferred_element_type=jnp.float32)


def _matmul(x, w):
    bm = 2048
    return pl.pallas_call(
        _mm_body,
        grid=(NP // bm,),
        in_specs=[pl.BlockSpec((bm, D), lambda i: (i, 0)),
                  pl.BlockSpec((D, D), lambda i: (0, 0))],
        out_specs=pl.BlockSpec((bm, D), lambda i: (i, 0)),
        out_shape=jax.ShapeDtypeStruct((NP, D), jnp.float32),
    )(x, w)


def _fin_body(x_ref, w_ref, b_ref, p_ref, o_ref):
    w = w_ref[...]
    ii = lax.broadcasted_iota(jnp.int32, (D, D), 0)
    jj = lax.broadcasted_iota(jnp.int32, (D, D), 1)
    gi = jnp.where(ii == jj, jnp.float32(GAMMA), jnp.float32(0.0))
    a = w.T - w - gi
    xa = jnp.dot(x_ref[...], a, preferred_element_type=jnp.float32)
    h = xa + p_ref[0] + p_ref[1] + b_ref[...]
    o_ref[...] = x_ref[...] + EPS * jnp.tanh(h)


def _finish(x, w, bias2d, partials):
    bm = 2048
    return pl.pallas_call(
        _fin_body,
        grid=(NP // bm,),
        in_specs=[pl.BlockSpec((bm, D), lambda i: (i, 0)),
                  pl.BlockSpec((D, D), lambda i: (0, 0)),
                  pl.BlockSpec((1, D), lambda i: (0, 0)),
                  pl.BlockSpec((NC, bm, D), lambda i: (0, i, 0))],
        out_specs=pl.BlockSpec((bm, D), lambda i: (i, 0)),
        out_shape=jax.ShapeDtypeStruct((N, D), jnp.float32),
    )(x, w, bias2d, partials)


def _rsqrt16(v):
    s = 0.5 * (v + 1.0)
    for _ in range(9):
        s = 0.5 * (s + v / s)
    return 1.0 / s


def _graph_body(xw_hbm, src_hbm, dst_hbm, tidx_hbm, out_hbm,
                y_hbm, deg_sh, acc_sh,
                sidx_v, didx_v, rbuf_v, gbuf_v,
                degv, dinvv, ones_v, sem, sem2):
    c = lax.axis_index("c")
    s = lax.axis_index("s")
    zero16 = jnp.zeros((L,), jnp.float32)

    with jax.named_scope("ph_a_zero"):
        for i in range(DSLC // L):
            degv[pl.ds(i * L, L)] = zero16

        def _zrow(r, _):
            for k in range(D // L):
                rbuf_v[r, pl.ds(k * L, L)] = zero16
            return 0

        lax.fori_loop(0, RCHUNK, _zrow, 0)
        pltpu.async_copy(degv.at[pl.ds(0, DSLC)], deg_sh.at[pl.ds(s * DSLC, DSLC)], sem)
        for t in range(NRC):
            pltpu.async_copy(rbuf_v,
                             acc_sh.at[pl.ds(s * RPS + t * RCHUNK, RCHUNK)],
                             sem)
        for i in range(CHUNK // L):
            ones_v[pl.ds(i * L, L)] = jnp.ones((L,), jnp.float32)
        pltpu.make_async_copy(degv.at[pl.ds(0, DSLC)],
                              deg_sh.at[pl.ds(s * DSLC, DSLC)], sem).wait()
        for t in range(NRC):
            pltpu.make_async_copy(
                rbuf_v, acc_sh.at[pl.ds(s * RPS + t * RCHUNK, RCHUNK)],
                sem).wait()
        plsc.subcore_barrier()

    with jax.named_scope("ph_b_hist"):
        def _hb_start(j):
            pltpu.async_copy(ones_v, deg_sh.at[sidx_v.at[j]], sem, add=True)

        def _hb_wait(j):
            pltpu.make_async_copy(ones_v, deg_sh.at[sidx_v.at[j]], sem).wait()

        def _hist(j, _):
            _hb_start(j)

            @pl.when(j >= 8)
            def _():
                _hb_wait(j - 8)

            return 0

        for h in range(HROWS // IBUF):
            pltpu.sync_copy(dst_hbm.at[pl.ds(s * HROWS + h * IBUF, IBUF)],
                            sidx_v)
            lax.fori_loop(0, IBUF, _hist, 0)
            for k in range(8):
                _hb_wait(k)
        t0 = (s // 4) * 8
        pltpu.sync_copy(tidx_hbm.at[pl.ds(t0, 8)], sidx_v.at[pl.ds(0, 8)])
        r0 = 2 * s - t0
        for q in range(2):
            pltpu.async_copy(ones_v, deg_sh.at[sidx_v.at[r0 + q]], sem,
                             add=True)
        for q in range(2):
            pltpu.make_async_copy(ones_v, deg_sh.at[sidx_v.at[r0 + q]],
                                  sem).wait()
        plsc.subcore_barrier()

    with jax.named_scope("ph_c_scale"):
        pltpu.sync_copy(deg_sh.at[pl.ds(s * DSLC, DSLC)], degv.at[pl.ds(0, DSLC)])
        for i in range(DSLC // L):
            dinvv[pl.ds(i * L, L)] = _rsqrt16(degv[pl.ds(i * L, L)])
        bufs = (rbuf_v, gbuf_v)
        sems = (sem, sem2)

        def _xw_start(t, b):
            pltpu.async_copy(xw_hbm.at[pl.ds(s * RPS + t * RCHUNK, RCHUNK)],
                             bufs[b], sems[b])

        def _xw_wait(t, b):
            pltpu.make_async_copy(
                xw_hbm.at[pl.ds(s * RPS + t * RCHUNK, RCHUNK)],
                bufs[b], sems[b]).wait()

        _xw_start(0, 0)
        _xw_start(1, 1)
        for t in range(NRC):
            b = t % 2
            buf = bufs[b]
            base = s * RPS + t * RCHUNK
            _xw_wait(t, b)

            def _scale(r, _, buf=buf, t=t):
                d = dinvv[pl.ds(t * RCHUNK + r, L)][0]
                for k in range(D // L):
                    sl = pl.ds(k * L, L)
                    buf[r, sl] = buf[r, sl] * d
                return 0

            lax.fori_loop(0, RCHUNK - 1, _scale, 0)
            dlast = dinvv[pl.ds(t * RCHUNK + RCHUNK - L, L)][L - 1]
            for k in range(D // L):
                sl = pl.ds(k * L, L)
                buf[RCHUNK - 1, sl] = buf[RCHUNK - 1, sl] * dlast
            pltpu.sync_copy(buf, y_hbm.at[c].at[pl.ds(base, RCHUNK)])
            if t + 2 < NRC:
                _xw_start(t + 2, b)
        plsc.subcore_barrier()

    with jax.named_scope("ph_d_edge"):
        row0 = (c * NS + s) * NCHUNK

        def _gstart(j, buf, gsem):
            pltpu.async_copy(y_hbm.at[c].at[sidx_v.at[j]], buf, gsem)

        def _gwait(j, buf, gsem):
            pltpu.make_async_copy(y_hbm.at[c].at[sidx_v.at[j]], buf,
                                  gsem).wait()

        for h in range(NCHUNK // IBUF):
            pltpu.sync_copy(src_hbm.at[pl.ds(row0 + h * IBUF, IBUF)], sidx_v)
            pltpu.sync_copy(dst_hbm.at[pl.ds(row0 + h * IBUF, IBUF)], didx_v)
            _gstart(0, gbuf_v, sem)
            _gstart(1, rbuf_v, sem2)

            def _edge2(jj, _):
                j0 = 2 * jj
                _gwait(j0, gbuf_v, sem)
                pltpu.sync_copy(gbuf_v, acc_sh.at[didx_v.at[j0]], add=True)
                _gstart(jnp.minimum(j0 + 2, IBUF - 1), gbuf_v, sem)
                j1 = j0 + 1
                _gwait(j1, rbuf_v, sem2)
                pltpu.sync_copy(rbuf_v, acc_sh.at[didx_v.at[j1]], add=True)
                _gstart(jnp.minimum(j1 + 2, IBUF - 1), rbuf_v, sem2)
                return 0

            lax.fori_loop(0, IBUF // 2, _edge2, 0)
            _gwait(IBUF - 1, gbuf_v, sem)
            _gwait(IBUF - 1, rbuf_v, sem2)
        w = c * NS + s
        tw = (w // 8) * 8
        pltpu.sync_copy(tidx_hbm.at[pl.ds(tw, 8)], sidx_v.at[pl.ds(0, 8)])
        rw = w - tw
        pltpu.async_copy(y_hbm.at[c].at[sidx_v.at[rw]], gbuf_v, sem)
        pltpu.make_async_copy(y_hbm.at[c].at[sidx_v.at[rw]], gbuf_v,
                              sem).wait()
        pltpu.sync_copy(gbuf_v, acc_sh.at[sidx_v.at[rw]], add=True)
        plsc.subcore_barrier()

    with jax.named_scope("ph_e_stage"):
        pltpu.async_copy(acc_sh.at[pl.ds(s * RPS, RCHUNK)], rbuf_v, sem)
        for t in range(NRC):
            base = s * RPS + t * RCHUNK
            pltpu.make_async_copy(acc_sh.at[pl.ds(base, RCHUNK)], rbuf_v,
                                  sem).wait()

            def _fin(r, _):
                d = dinvv[pl.ds(t * RCHUNK + r, L)][0]
                for k in range(D // L):
                    sl = pl.ds(k * L, L)
                    gbuf_v[r, sl] = d * rbuf_v[r, sl]
                return 0

            lax.fori_loop(0, RCHUNK - 1, _fin, 0)
            dlast = dinvv[pl.ds(t * RCHUNK + RCHUNK - L, L)][L - 1]
            for k in range(D // L):
                sl = pl.ds(k * L, L)
                gbuf_v[RCHUNK - 1, sl] = dlast * rbuf_v[RCHUNK - 1, sl]
            if t + 1 < NRC:
                pltpu.async_copy(acc_sh.at[pl.ds(base + RCHUNK, RCHUNK)],
                                 rbuf_v, sem)
            pltpu.sync_copy(gbuf_v, out_hbm.at[c].at[pl.ds(base, RCHUNK)])


def _graph_sc(xw, src2d, dst2d, tail2d):
    mesh = plsc.VectorSubcoreMesh(core_axis_name="c", subcore_axis_name="s")
    kfn = pl.kernel(
        _graph_body,
        out_type=jax.ShapeDtypeStruct((NC, NP, D), jnp.float32),
        mesh=mesh,
        scratch_types=[
            pltpu.HBM((NC, NP, D), jnp.float32),
            pltpu.VMEM_SHARED((NP,), jnp.float32),
            pltpu.VMEM_SHARED((NP, D), jnp.float32),
            pltpu.VMEM((IBUF, CHUNK), jnp.int32),
            pltpu.VMEM((IBUF, CHUNK), jnp.int32),
            pltpu.VMEM((RCHUNK, D), jnp.float32),
            pltpu.VMEM((CHUNK, D), jnp.float32),
            pltpu.VMEM((DSLC + L,), jnp.float32),
            pltpu.VMEM((DSLC + L,), jnp.float32),
            pltpu.VMEM((CHUNK,), jnp.float32),
            pltpu.SemaphoreType.DMA,
            pltpu.SemaphoreType.DMA,
        ],
    )
    return kfn(xw, src2d, dst2d, tail2d)


@jax.jit
def kernel(x, edge_index, W, bias, W_phi):
    self_idx = jnp.arange(SELF0, dtype=jnp.int32)
    src2d = jnp.concatenate([edge_index[0], self_idx]).reshape(EROWS, CHUNK)
    dst2d = jnp.concatenate([edge_index[1], self_idx]).reshape(EROWS, CHUNK)
    tail_self = SELF0 + jnp.arange(N - SELF0, dtype=jnp.int32)
    tail_pad = N + jnp.arange(TROWS * CHUNK - (N - SELF0),
                              dtype=jnp.int32) % (NP - N)
    tail2d = jnp.concatenate([tail_self, tail_pad]).reshape(TROWS, CHUNK)
    x_pad = jnp.pad(x, ((0, NP - N), (0, 0)))
    xw = _matmul(x_pad, W_phi)
    partials = _graph_sc(xw, src2d, dst2d, tail2d)
    return _finish(x_pad, W, bias.reshape(1, D), partials)


# speedup vs baseline: 1.2223x; 1.0087x over previous
import jax
import jax.numpy as jnp
from jax import lax
from jax.experimental import pallas as pl
from jax.experimental.pallas import tpu as pltpu
from jax.experimental.pallas import tpu_sc as plsc

GAMMA = 0.1
EPS = 0.1

N = 10000
E = 320000
D = 128

NC = 2
NS = 16
L = 16

NP = 10240
EP = 327680
CHUNK = 128
EROWS = EP // CHUNK
EPW = EP // (NC * NS)
NCHUNK = EPW // CHUNK
HROWS = EROWS // NS
IBUF = 40
SELF0 = EP - E
TROWS = 32

RPS = NP // NS
RCHUNK = 128
NRC = RPS // RCHUNK
DSLC = NP // NS


def _mm_body(x_ref, w_ref, o_ref):
    o_ref[...] = jnp.dot(x_ref[...], w_ref[...],
                         preferred_element_type=jnp.float32)


def _matmul(x, w):
    bm = 2560
    return pl.pallas_call(
        _mm_body,
        grid=(NP // bm,),
        in_specs=[pl.BlockSpec((bm, D), lambda i: (i, 0)),
                  pl.BlockSpec((D, D), lambda i: (0, 0))],
        out_specs=pl.BlockSpec((bm, D), lambda i: (i, 0)),
        out_shape=jax.ShapeDtypeStruct((NP, D), jnp.float32),
    )(x, w)


def _fin_body(x_ref, w_ref, b_ref, p_ref, o_ref):
    w = w_ref[...]
    ii = lax.broadcasted_iota(jnp.int32, (D, D), 0)
    jj = lax.broadcasted_iota(jnp.int32, (D, D), 1)
    gi = jnp.where(ii == jj, jnp.float32(GAMMA), jnp.float32(0.0))
    a = w.T - w - gi
    xa = jnp.dot(x_ref[...], a, preferred_element_type=jnp.float32)
    h = xa + p_ref[0] + p_ref[1] + b_ref[...]
    o_ref[...] = x_ref[...] + EPS * jnp.tanh(h)


def _finish(x, w, bias2d, partials):
    bm = 2560
    return pl.pallas_call(
        _fin_body,
        grid=(NP // bm,),
        in_specs=[pl.BlockSpec((bm, D), lambda i: (i, 0)),
                  pl.BlockSpec((D, D), lambda i: (0, 0)),
                  pl.BlockSpec((1, D), lambda i: (0, 0)),
                  pl.BlockSpec((NC, bm, D), lambda i: (0, i, 0))],
        out_specs=pl.BlockSpec((bm, D), lambda i: (i, 0)),
        out_shape=jax.ShapeDtypeStruct((N, D), jnp.float32),
    )(x, w, bias2d, partials)


def _rsqrt16(v):
    s = 0.5 * (v + 1.0)
    for _ in range(9):
        s = 0.5 * (s + v / s)
    return 1.0 / s


def _graph_body(xw_hbm, src_hbm, dst_hbm, tidx_hbm, out_hbm,
                y_hbm, deg_sh, acc_sh,
                sidx_v, didx_v, rbuf_v, gbuf_v,
                degv, dinvv, ones_v, sem, sem2):
    c = lax.axis_index("c")
    s = lax.axis_index("s")
    zero16 = jnp.zeros((L,), jnp.float32)

    with jax.named_scope("ph_a_zero"):
        for i in range(DSLC // L):
            degv[pl.ds(i * L, L)] = zero16

        def _zrow(r, _):
            for k in range(D // L):
                rbuf_v[r, pl.ds(k * L, L)] = zero16
            return 0

        lax.fori_loop(0, RCHUNK, _zrow, 0)
        pltpu.async_copy(degv.at[pl.ds(0, DSLC)], deg_sh.at[pl.ds(s * DSLC, DSLC)], sem)
        for t in range(NRC):
            pltpu.async_copy(rbuf_v,
                             acc_sh.at[pl.ds(s * RPS + t * RCHUNK, RCHUNK)],
                             sem)
        for i in range(CHUNK // L):
            ones_v[pl.ds(i * L, L)] = jnp.ones((L,), jnp.float32)
        pltpu.make_async_copy(degv.at[pl.ds(0, DSLC)],
                              deg_sh.at[pl.ds(s * DSLC, DSLC)], sem).wait()
        for t in range(NRC):
            pltpu.make_async_copy(
                rbuf_v, acc_sh.at[pl.ds(s * RPS + t * RCHUNK, RCHUNK)],
                sem).wait()
        plsc.subcore_barrier()

    with jax.named_scope("ph_b_hist"):
        def _hb_start(j):
            pltpu.async_copy(ones_v, deg_sh.at[sidx_v.at[j]], sem, add=True)

        def _hb_wait(j):
            pltpu.make_async_copy(ones_v, deg_sh.at[sidx_v.at[j]], sem).wait()

        def _hist(j, _):
            _hb_start(j)

            @pl.when(j >= 8)
            def _():
                _hb_wait(j - 8)

            return 0

        for h in range(HROWS // IBUF):
            pltpu.sync_copy(dst_hbm.at[pl.ds(s * HROWS + h * IBUF, IBUF)],
                            sidx_v)
            lax.fori_loop(0, IBUF, _hist, 0)
            for k in range(8):
                _hb_wait(k)
        t0 = (s // 4) * 8
        pltpu.sync_copy(tidx_hbm.at[pl.ds(t0, 8)], sidx_v.at[pl.ds(0, 8)])
        r0 = 2 * s - t0
        for q in range(2):
            pltpu.async_copy(ones_v, deg_sh.at[sidx_v.at[r0 + q]], sem,
                             add=True)
        for q in range(2):
            pltpu.make_async_copy(ones_v, deg_sh.at[sidx_v.at[r0 + q]],
                                  sem).wait()
        plsc.subcore_barrier()

    with jax.named_scope("ph_c_scale"):
        pltpu.sync_copy(deg_sh.at[pl.ds(s * DSLC, DSLC)], degv.at[pl.ds(0, DSLC)])
        for i in range(DSLC // L):
            dinvv[pl.ds(i * L, L)] = _rsqrt16(degv[pl.ds(i * L, L)])
        bufs = (rbuf_v, gbuf_v)
        sems = (sem, sem2)

        def _xw_start(t, b):
            pltpu.async_copy(xw_hbm.at[pl.ds(s * RPS + t * RCHUNK, RCHUNK)],
                             bufs[b], sems[b])

        def _xw_wait(t, b):
            pltpu.make_async_copy(
                xw_hbm.at[pl.ds(s * RPS + t * RCHUNK, RCHUNK)],
                bufs[b], sems[b]).wait()

        _xw_start(0, 0)
        _xw_start(1, 1)
        for t in range(NRC):
            b = t % 2
            buf = bufs[b]
            base = s * RPS + t * RCHUNK
            _xw_wait(t, b)

            def _scale(r, _, buf=buf, t=t):
                d = dinvv[pl.ds(t * RCHUNK + r, L)][0]
                for k in range(D // L):
                    sl = pl.ds(k * L, L)
                    buf[r, sl] = buf[r, sl] * d
                return 0

            lax.fori_loop(0, RCHUNK - 1, _scale, 0)
            dlast = dinvv[pl.ds(t * RCHUNK + RCHUNK - L, L)][L - 1]
            for k in range(D // L):
                sl = pl.ds(k * L, L)
                buf[RCHUNK - 1, sl] = buf[RCHUNK - 1, sl] * dlast
            pltpu.sync_copy(buf, y_hbm.at[c].at[pl.ds(base, RCHUNK)])
            if t + 2 < NRC:
                _xw_start(t + 2, b)
        plsc.subcore_barrier()

    with jax.named_scope("ph_d_edge"):
        row0 = (c * NS + s) * NCHUNK

        def _gstart(j, buf, gsem):
            pltpu.async_copy(y_hbm.at[c].at[sidx_v.at[j]], buf, gsem)

        def _gwait(j, buf, gsem):
            pltpu.make_async_copy(y_hbm.at[c].at[sidx_v.at[j]], buf,
                                  gsem).wait()

        for h in range(NCHUNK // IBUF):
            pltpu.sync_copy(src_hbm.at[pl.ds(row0 + h * IBUF, IBUF)], sidx_v)
            pltpu.sync_copy(dst_hbm.at[pl.ds(row0 + h * IBUF, IBUF)], didx_v)
            _gstart(0, gbuf_v, sem)
            _gstart(1, rbuf_v, sem2)

            def _edge2(jj, _):
                j0 = 2 * jj
                _gwait(j0, gbuf_v, sem)
                pltpu.sync_copy(gbuf_v, acc_sh.at[didx_v.at[j0]], add=True)
                _gstart(jnp.minimum(j0 + 2, IBUF - 1), gbuf_v, sem)
                j1 = j0 + 1
                _gwait(j1, rbuf_v, sem2)
                pltpu.sync_copy(rbuf_v, acc_sh.at[didx_v.at[j1]], add=True)
                _gstart(jnp.minimum(j1 + 2, IBUF - 1), rbuf_v, sem2)
                return 0

            lax.fori_loop(0, IBUF // 2, _edge2, 0)
            _gwait(IBUF - 1, gbuf_v, sem)
            _gwait(IBUF - 1, rbuf_v, sem2)
        w = c * NS + s
        tw = (w // 8) * 8
        pltpu.sync_copy(tidx_hbm.at[pl.ds(tw, 8)], sidx_v.at[pl.ds(0, 8)])
        rw = w - tw
        pltpu.async_copy(y_hbm.at[c].at[sidx_v.at[rw]], gbuf_v, sem)
        pltpu.make_async_copy(y_hbm.at[c].at[sidx_v.at[rw]], gbuf_v,
                              sem).wait()
        pltpu.sync_copy(gbuf_v, acc_sh.at[sidx_v.at[rw]], add=True)
        plsc.subcore_barrier()

    with jax.named_scope("ph_e_stage"):
        pltpu.async_copy(acc_sh.at[pl.ds(s * RPS, RCHUNK)], rbuf_v, sem)
        for t in range(NRC):
            base = s * RPS + t * RCHUNK
            pltpu.make_async_copy(acc_sh.at[pl.ds(base, RCHUNK)], rbuf_v,
                                  sem).wait()

            def _fin(r, _):
                d = dinvv[pl.ds(t * RCHUNK + r, L)][0]
                for k in range(D // L):
                    sl = pl.ds(k * L, L)
                    gbuf_v[r, sl] = d * rbuf_v[r, sl]
                return 0

            lax.fori_loop(0, RCHUNK - 1, _fin, 0)
            dlast = dinvv[pl.ds(t * RCHUNK + RCHUNK - L, L)][L - 1]
            for k in range(D // L):
                sl = pl.ds(k * L, L)
                gbuf_v[RCHUNK - 1, sl] = dlast * rbuf_v[RCHUNK - 1, sl]
            if t + 1 < NRC:
                pltpu.async_copy(acc_sh.at[pl.ds(base + RCHUNK, RCHUNK)],
                                 rbuf_v, sem)
            pltpu.sync_copy(gbuf_v, out_hbm.at[c].at[pl.ds(base, RCHUNK)])


def _graph_sc(xw, src2d, dst2d, tail2d):
    mesh = plsc.VectorSubcoreMesh(core_axis_name="c", subcore_axis_name="s")
    kfn = pl.kernel(
        _graph_body,
        out_type=jax.ShapeDtypeStruct((NC, NP, D), jnp.float32),
        mesh=mesh,
        scratch_types=[
            pltpu.HBM((NC, NP, D), jnp.float32),
            pltpu.VMEM_SHARED((NP,), jnp.float32),
            pltpu.VMEM_SHARED((NP, D), jnp.float32),
            pltpu.VMEM((IBUF, CHUNK), jnp.int32),
            pltpu.VMEM((IBUF, CHUNK), jnp.int32),
            pltpu.VMEM((RCHUNK, D), jnp.float32),
            pltpu.VMEM((CHUNK, D), jnp.float32),
            pltpu.VMEM((DSLC + L,), jnp.float32),
            pltpu.VMEM((DSLC + L,), jnp.float32),
            pltpu.VMEM((CHUNK,), jnp.float32),
            pltpu.SemaphoreType.DMA,
            pltpu.SemaphoreType.DMA,
        ],
    )
    return kfn(xw, src2d, dst2d, tail2d)


@jax.jit
def kernel(x, edge_index, W, bias, W_phi):
    self_idx = jnp.arange(SELF0, dtype=jnp.int32)
    src2d = jnp.concatenate([edge_index[0], self_idx]).reshape(EROWS, CHUNK)
    dst2d = jnp.concatenate([edge_index[1], self_idx]).reshape(EROWS, CHUNK)
    tail_self = SELF0 + jnp.arange(N - SELF0, dtype=jnp.int32)
    tail_pad = N + jnp.arange(TROWS * CHUNK - (N - SELF0),
                              dtype=jnp.int32) % (NP - N)
    tail2d = jnp.concatenate([tail_self, tail_pad]).reshape(TROWS, CHUNK)
    x_pad = jnp.pad(x, ((0, NP - N), (0, 0)))
    xw = _matmul(x_pad, W_phi)
    partials = _graph_sc(xw, src2d, dst2d, tail2d)
    return _finish(x_pad, W, bias.reshape(1, D), partials)


# speedup vs baseline: 1.2417x; 1.0159x over previous
import jax
import jax.numpy as jnp
from jax import lax
from jax.experimental import pallas as pl
from jax.experimental.pallas import tpu as pltpu
from jax.experimental.pallas import tpu_sc as plsc

GAMMA = 0.1
EPS = 0.1

N = 10000
E = 320000
D = 128

NC = 2
NS = 16
L = 16

NP = 10240
EP = 327680
CHUNK = 128
EROWS = EP // CHUNK
EPW = EP // (NC * NS)
NCHUNK = EPW // CHUNK
HROWS = EROWS // NS
IBUF = 40
SELF0 = EP - E
TROWS = 32

RPS = NP // NS
RCHUNK = 128
NRC = RPS // RCHUNK
DSLC = NP // NS


def _mm_body(x_ref, w_ref, o_ref):
    o_ref[...] = jnp.dot(x_ref[...], w_ref[...],
                         preferred_element_type=jnp.float32)


def _matmul(x, w):
    bm = 5120
    return pl.pallas_call(
        _mm_body,
        grid=(NP // bm,),
        in_specs=[pl.BlockSpec((bm, D), lambda i: (i, 0)),
                  pl.BlockSpec((D, D), lambda i: (0, 0))],
        out_specs=pl.BlockSpec((bm, D), lambda i: (i, 0)),
        out_shape=jax.ShapeDtypeStruct((NP, D), jnp.float32),
    )(x, w)


def _fin_body(x_ref, w_ref, b_ref, p_ref, o_ref):
    w = w_ref[...]
    ii = lax.broadcasted_iota(jnp.int32, (D, D), 0)
    jj = lax.broadcasted_iota(jnp.int32, (D, D), 1)
    gi = jnp.where(ii == jj, jnp.float32(GAMMA), jnp.float32(0.0))
    a = w.T - w - gi
    xa = jnp.dot(x_ref[...], a, preferred_element_type=jnp.float32)
    h = xa + p_ref[0] + p_ref[1] + b_ref[...]
    o_ref[...] = x_ref[...] + EPS * jnp.tanh(h)


def _finish(x, w, bias2d, partials):
    bm = 5120
    return pl.pallas_call(
        _fin_body,
        grid=(NP // bm,),
        in_specs=[pl.BlockSpec((bm, D), lambda i: (i, 0)),
                  pl.BlockSpec((D, D), lambda i: (0, 0)),
                  pl.BlockSpec((1, D), lambda i: (0, 0)),
                  pl.BlockSpec((NC, bm, D), lambda i: (0, i, 0))],
        out_specs=pl.BlockSpec((bm, D), lambda i: (i, 0)),
        out_shape=jax.ShapeDtypeStruct((N, D), jnp.float32),
    )(x, w, bias2d, partials)


def _rsqrt16(v):
    s = 0.5 * (v + 1.0)
    for _ in range(9):
        s = 0.5 * (s + v / s)
    return 1.0 / s


def _graph_body(xw_hbm, src_hbm, dst_hbm, tidx_hbm, out_hbm,
                y_hbm, deg_sh, acc_sh,
                sidx_v, didx_v, rbuf_v, gbuf_v,
                degv, dinvv, ones_v, sem, sem2):
    c = lax.axis_index("c")
    s = lax.axis_index("s")
    zero16 = jnp.zeros((L,), jnp.float32)

    with jax.named_scope("ph_a_zero"):
        for i in range(DSLC // L):
            degv[pl.ds(i * L, L)] = zero16

        def _zrow(r, _):
            for k in range(D // L):
                rbuf_v[r, pl.ds(k * L, L)] = zero16
            return 0

        lax.fori_loop(0, RCHUNK, _zrow, 0)
        pltpu.async_copy(degv.at[pl.ds(0, DSLC)], deg_sh.at[pl.ds(s * DSLC, DSLC)], sem)
        for t in range(NRC):
            pltpu.async_copy(rbuf_v,
                             acc_sh.at[pl.ds(s * RPS + t * RCHUNK, RCHUNK)],
                             sem)
        for i in range(CHUNK // L):
            ones_v[pl.ds(i * L, L)] = jnp.ones((L,), jnp.float32)
        pltpu.make_async_copy(degv.at[pl.ds(0, DSLC)],
                              deg_sh.at[pl.ds(s * DSLC, DSLC)], sem).wait()
        for t in range(NRC):
            pltpu.make_async_copy(
                rbuf_v, acc_sh.at[pl.ds(s * RPS + t * RCHUNK, RCHUNK)],
                sem).wait()
        plsc.subcore_barrier()

    with jax.named_scope("ph_b_hist"):
        def _hb_start(j):
            pltpu.async_copy(ones_v, deg_sh.at[sidx_v.at[j]], sem, add=True)

        def _hb_wait(j):
            pltpu.make_async_copy(ones_v, deg_sh.at[sidx_v.at[j]], sem).wait()

        def _hist(j, _):
            _hb_start(j)

            @pl.when(j >= 8)
            def _():
                _hb_wait(j - 8)

            return 0

        for h in range(HROWS // IBUF):
            pltpu.sync_copy(dst_hbm.at[pl.ds(s * HROWS + h * IBUF, IBUF)],
                            sidx_v)
            lax.fori_loop(0, IBUF, _hist, 0)
            for k in range(8):
                _hb_wait(k)
        t0 = (s // 4) * 8
        pltpu.sync_copy(tidx_hbm.at[pl.ds(t0, 8)], sidx_v.at[pl.ds(0, 8)])
        r0 = 2 * s - t0
        for q in range(2):
            pltpu.async_copy(ones_v, deg_sh.at[sidx_v.at[r0 + q]], sem,
                             add=True)
        for q in range(2):
            pltpu.make_async_copy(ones_v, deg_sh.at[sidx_v.at[r0 + q]],
                                  sem).wait()
        plsc.subcore_barrier()

    with jax.named_scope("ph_c_scale"):
        pltpu.sync_copy(deg_sh.at[pl.ds(s * DSLC, DSLC)], degv.at[pl.ds(0, DSLC)])
        for i in range(DSLC // L):
            dinvv[pl.ds(i * L, L)] = _rsqrt16(degv[pl.ds(i * L, L)])
        bufs = (rbuf_v, gbuf_v)
        sems = (sem, sem2)

        def _xw_start(t, b):
            pltpu.async_copy(xw_hbm.at[pl.ds(s * RPS + t * RCHUNK, RCHUNK)],
                             bufs[b], sems[b])

        def _xw_wait(t, b):
            pltpu.make_async_copy(
                xw_hbm.at[pl.ds(s * RPS + t * RCHUNK, RCHUNK)],
                bufs[b], sems[b]).wait()

        _xw_start(0, 0)
        _xw_start(1, 1)
        for t in range(NRC):
            b = t % 2
            buf = bufs[b]
            base = s * RPS + t * RCHUNK
            _xw_wait(t, b)

            def _scale(r, _, buf=buf, t=t):
                d = dinvv[pl.ds(t * RCHUNK + r, L)][0]
                for k in range(D // L):
                    sl = pl.ds(k * L, L)
                    buf[r, sl] = buf[r, sl] * d
                return 0

            lax.fori_loop(0, RCHUNK - 1, _scale, 0)
            dlast = dinvv[pl.ds(t * RCHUNK + RCHUNK - L, L)][L - 1]
            for k in range(D // L):
                sl = pl.ds(k * L, L)
                buf[RCHUNK - 1, sl] = buf[RCHUNK - 1, sl] * dlast
            pltpu.sync_copy(buf, y_hbm.at[c].at[pl.ds(base, RCHUNK)])
            if t + 2 < NRC:
                _xw_start(t + 2, b)
        plsc.subcore_barrier()

    with jax.named_scope("ph_d_edge"):
        row0 = (c * NS + s) * NCHUNK

        def _gstart(j, buf, gsem):
            pltpu.async_copy(y_hbm.at[c].at[sidx_v.at[j]], buf, gsem)

        def _gwait(j, buf, gsem):
            pltpu.make_async_copy(y_hbm.at[c].at[sidx_v.at[j]], buf,
                                  gsem).wait()

        for h in range(NCHUNK // IBUF):
            pltpu.sync_copy(src_hbm.at[pl.ds(row0 + h * IBUF, IBUF)], sidx_v)
            pltpu.sync_copy(dst_hbm.at[pl.ds(row0 + h * IBUF, IBUF)], didx_v)
            _gstart(0, gbuf_v, sem)
            _gstart(1, rbuf_v, sem2)

            def _edge2(jj, _):
                j0 = 2 * jj
                _gwait(j0, gbuf_v, sem)
                pltpu.sync_copy(gbuf_v, acc_sh.at[didx_v.at[j0]], add=True)
                _gstart(jnp.minimum(j0 + 2, IBUF - 1), gbuf_v, sem)
                j1 = j0 + 1
                _gwait(j1, rbuf_v, sem2)
                pltpu.sync_copy(rbuf_v, acc_sh.at[didx_v.at[j1]], add=True)
                _gstart(jnp.minimum(j1 + 2, IBUF - 1), rbuf_v, sem2)
                return 0

            lax.fori_loop(0, IBUF // 2, _edge2, 0)
            _gwait(IBUF - 1, gbuf_v, sem)
            _gwait(IBUF - 1, rbuf_v, sem2)
        w = c * NS + s
        tw = (w // 8) * 8
        pltpu.sync_copy(tidx_hbm.at[pl.ds(tw, 8)], sidx_v.at[pl.ds(0, 8)])
        rw = w - tw
        pltpu.async_copy(y_hbm.at[c].at[sidx_v.at[rw]], gbuf_v, sem)
        pltpu.make_async_copy(y_hbm.at[c].at[sidx_v.at[rw]], gbuf_v,
                              sem).wait()
        pltpu.sync_copy(gbuf_v, acc_sh.at[sidx_v.at[rw]], add=True)
        plsc.subcore_barrier()

    with jax.named_scope("ph_e_stage"):
        pltpu.async_copy(acc_sh.at[pl.ds(s * RPS, RCHUNK)], rbuf_v, sem)
        for t in range(NRC):
            base = s * RPS + t * RCHUNK
            pltpu.make_async_copy(acc_sh.at[pl.ds(base, RCHUNK)], rbuf_v,
                                  sem).wait()

            def _fin(r, _):
                d = dinvv[pl.ds(t * RCHUNK + r, L)][0]
                for k in range(D // L):
                    sl = pl.ds(k * L, L)
                    gbuf_v[r, sl] = d * rbuf_v[r, sl]
                return 0

            lax.fori_loop(0, RCHUNK - 1, _fin, 0)
            dlast = dinvv[pl.ds(t * RCHUNK + RCHUNK - L, L)][L - 1]
            for k in range(D // L):
                sl = pl.ds(k * L, L)
                gbuf_v[RCHUNK - 1, sl] = dlast * rbuf_v[RCHUNK - 1, sl]
            if t + 1 < NRC:
                pltpu.async_copy(acc_sh.at[pl.ds(base + RCHUNK, RCHUNK)],
                                 rbuf_v, sem)
            pltpu.sync_copy(gbuf_v, out_hbm.at[c].at[pl.ds(base, RCHUNK)])


def _graph_sc(xw, src2d, dst2d, tail2d):
    mesh = plsc.VectorSubcoreMesh(core_axis_name="c", subcore_axis_name="s")
    kfn = pl.kernel(
        _graph_body,
        out_type=jax.ShapeDtypeStruct((NC, NP, D), jnp.float32),
        mesh=mesh,
        scratch_types=[
            pltpu.HBM((NC, NP, D), jnp.float32),
            pltpu.VMEM_SHARED((NP,), jnp.float32),
            pltpu.VMEM_SHARED((NP, D), jnp.float32),
            pltpu.VMEM((IBUF, CHUNK), jnp.int32),
            pltpu.VMEM((IBUF, CHUNK), jnp.int32),
            pltpu.VMEM((RCHUNK, D), jnp.float32),
            pltpu.VMEM((CHUNK, D), jnp.float32),
            pltpu.VMEM((DSLC + L,), jnp.float32),
            pltpu.VMEM((DSLC + L,), jnp.float32),
            pltpu.VMEM((CHUNK,), jnp.float32),
            pltpu.SemaphoreType.DMA,
            pltpu.SemaphoreType.DMA,
        ],
    )
    return kfn(xw, src2d, dst2d, tail2d)


@jax.jit
def kernel(x, edge_index, W, bias, W_phi):
    self_idx = jnp.arange(SELF0, dtype=jnp.int32)
    src2d = jnp.concatenate([edge_index[0], self_idx]).reshape(EROWS, CHUNK)
    dst2d = jnp.concatenate([edge_index[1], self_idx]).reshape(EROWS, CHUNK)
    tail_self = SELF0 + jnp.arange(N - SELF0, dtype=jnp.int32)
    tail_pad = N + jnp.arange(TROWS * CHUNK - (N - SELF0),
                              dtype=jnp.int32) % (NP - N)
    tail2d = jnp.concatenate([tail_self, tail_pad]).reshape(TROWS, CHUNK)
    x_pad = jnp.pad(x, ((0, NP - N), (0, 0)))
    xw = _matmul(x_pad, W_phi)
    partials = _graph_sc(xw, src2d, dst2d, tail2d)
    return _finish(x_pad, W, bias.reshape(1, D), partials)
